# baseline (speedup 1.0000x reference)
import functools

import jax
import jax.numpy as jnp
from jax import lax
from jax.experimental import pallas as pl
from jax.experimental.pallas import tpu as pltpu
from jax.experimental.pallas import tpu_sc as plsc

B = 8
N_TOKEN = 512
N_CHAR = 8
FLAT = N_TOKEN * N_CHAR
SEQ = 2048
TEXT_D = 512
TOK_D = 768
EPS = 1e-6
ROWS = B * 32
BUFW = FLAT + 16
HALF_D = TEXT_D // 2

_info = plsc.get_sparse_core_info()
NC, NS = _info.num_cores, _info.num_subcores
NW = NC * NS

_MESH = plsc.VectorSubcoreMesh(core_axis_name="c", subcore_axis_name="s")
_HI = lax.Precision.HIGHEST


def _bf16_bits(x):
    u = jax.lax.bitcast_convert_type(x, jnp.int32)
    r = u + 0x7FFF + ((u >> 16) & 1)
    return (r >> 16) & 0xFFFF


def _tc_prep_body(mask_ref, cid_ref, text_ref, dest_ref, packed_ref, cnt_ref,
                  tbl_ref):
    f32 = jnp.float32
    i32 = jnp.int32
    mi = mask_ref[...]
    m2 = mi.astype(f32)
    a_ = lax.broadcasted_iota(i32, (ROWS, ROWS), 0)
    b_ = lax.broadcasted_iota(i32, (ROWS, ROWS), 1)
    lt_bd = ((b_ < a_) & (b_ // 32 == a_ // 32)).astype(f32)
    prev_rows = lax.dot_general(lt_bd, m2, (((1,), (0,)), ((), ())),
                                precision=_HI, preferred_element_type=f32)
    rowoff = jnp.sum(prev_rows, axis=-1, keepdims=True)
    cp = lax.broadcasted_iota(i32, (128, 128), 0)
    cc = lax.broadcasted_iota(i32, (128, 128), 1)
    lt_incl = (cp <= cc).astype(f32)
    incl = lax.dot_general(m2, lt_incl, (((1,), (0,)), ((), ())),
                           precision=_HI, preferred_element_type=f32) + rowoff
    row = lax.broadcasted_iota(i32, (ROWS, 128), 0)
    col = lax.broadcasted_iota(i32, (ROWS, 128), 1)
    base = lax.rem(row // 32, 4) * BUFW
    dest_ref[...] = base + jnp.where(mi != 0, incl.astype(i32) - 1,
                                     FLAT + lax.rem(col, 16))
    tok = (lax.rem(row, 32) * 128 + col) // N_CHAR
    packed_ref[...] = (cid_ref[...] + 1) + tok * 4096
    rowsum = jnp.sum(m2, axis=-1, keepdims=True)
    bb = lax.broadcasted_iota(i32, (B, ROWS), 0)
    rr = lax.broadcasted_iota(i32, (B, ROWS), 1)
    ind = (rr // 32 == bb).astype(f32)
    total = lax.dot_general(ind, rowsum, (((1,), (0,)), ((), ())),
                            precision=_HI, preferred_element_type=f32)
    cnt_ref[...] = jnp.broadcast_to(total.astype(i32), (B, 128))
    tx = text_ref[...]
    lo = _bf16_bits(tx[:, :HALF_D])
    hi = _bf16_bits(tx[:, HALF_D:])
    tbl_ref[...] = lo | (hi << 16)


def _tc_prep(mask2d, cid2d, text_embed):
    return pl.pallas_call(
        _tc_prep_body,
        out_shape=[
            jax.ShapeDtypeStruct((ROWS, 128), jnp.int32),
            jax.ShapeDtypeStruct((ROWS, 128), jnp.int32),
            jax.ShapeDtypeStruct((B, 128), jnp.int32),
            jax.ShapeDtypeStruct((SEQ + 1, HALF_D), jnp.int32),
        ],
    )(mask2d, cid2d, text_embed)


TOK_CHUNK = 64
TOK_NCHUNK = (B * N_TOKEN) // TOK_CHUNK
CHAR_CHUNK = 128
CHAR_NCHUNK = 512 // CHAR_CHUNK


@functools.partial(
    pl.kernel,
    mesh=_MESH,
    out_type=[
        jax.ShapeDtypeStruct((B * SEQ,), jnp.int32),
        jax.ShapeDtypeStruct((B * SEQ, HALF_D), jnp.int32),
        jax.ShapeDtypeStruct((B * N_TOKEN, TOK_D), jnp.float32),
    ],
    scratch_types=[
        pltpu.VMEM((32, 128), jnp.int32),
        pltpu.VMEM((32, 128), jnp.int32),
        pltpu.VMEM((CHAR_CHUNK,), jnp.int32),
        pltpu.VMEM((CHAR_CHUNK,), jnp.int32),
        pltpu.VMEM((CHAR_CHUNK, HALF_D), jnp.int32),
        pltpu.VMEM((CHAR_CHUNK, HALF_D), jnp.int32),
        pltpu.VMEM((TOK_CHUNK,), jnp.int32),
        pltpu.VMEM((TOK_CHUNK, TOK_D), jnp.float32),
        pltpu.VMEM((SEQ,), jnp.int32),
        pltpu.VMEM_SHARED((4 * BUFW,), jnp.int32),
        pltpu.SemaphoreType.DMA,
        pltpu.SemaphoreType.DMA,
    ],
)
def _sc_compact_gather(token_ids_hbm, packed_hbm, dest_hbm, tbl_hbm,
                       token_embed_hbm, ids_out, gath_out, temb_out,
                       vals_v, dst_v, idx0_v, idx1_v, rows0_v, rows1_v,
                       tidx_v, trows_v, bounce_v, shared_v, sem, sem_o):
    c = lax.axis_index("c")
    s = lax.axis_index("s")

    def token_chunk(ch):
        base = ch * TOK_CHUNK
        pltpu.sync_copy(token_ids_hbm.at[pl.ds(base, TOK_CHUNK)], tidx_v)
        pltpu.async_copy(token_embed_hbm.at[tidx_v], trows_v, sem).wait()
        pltpu.sync_copy(trows_v, temb_out.at[pl.ds(base, TOK_CHUNK)])

    @pl.when(s < 4)
    def _():
        b = c * 4 + s
        pltpu.sync_copy(packed_hbm.at[b], vals_v)
        pltpu.sync_copy(dest_hbm.at[b], dst_v)

        def scat_body(j4, carry):
            ds = []
            for p in range(8):
                j = j4 * 8 + p
                ds.append(pltpu.async_copy(
                    vals_v.at[j], shared_v.at[dst_v.at[j]], sem))
            for d in ds:
                d.wait()
            return carry

        lax.fori_loop(0, 4, scat_body, 0)
        token_chunk(56 + c * 4 + s)

    @pl.when(s >= 4)
    def _():
        t = c * 12 + (s - 4)
        token_chunk(t)
        token_chunk(t + 24)

        @pl.when(t < 8)
        def _():
            token_chunk(t + 48)

    plsc.subcore_barrier()

    b_loc = s // 4
    off = lax.rem(s, 4) * 512
    idx = [idx0_v, idx1_v]
    rows = [rows0_v, rows1_v]
    gat = [None] * CHAR_NCHUNK
    out = [None] * CHAR_NCHUNK

    for k in range(CHAR_NCHUNK):
        if k >= 2:
            out[k - 2].wait()
        src = b_loc * BUFW + off + k * CHAR_CHUNK
        ib = idx[k % 2]
        pltpu.sync_copy(shared_v.at[pl.ds(src, CHAR_CHUNK)], ib)
        for p in range(CHAR_CHUNK // 16):
            v = ib[pl.ds(p * 16, 16)]
            ib[pl.ds(p * 16, 16)] = jnp.clip(jnp.bitwise_and(v, 4095), 0, SEQ)
        gat[k] = pltpu.async_copy(tbl_hbm.at[ib], rows[k % 2], sem)
        if k >= 1:
            gat[k - 1].wait()
            dstp = (c * 4 + b_loc) * SEQ + off + (k - 1) * CHAR_CHUNK
            out[k - 1] = pltpu.async_copy(
                rows[(k - 1) % 2], gath_out.at[pl.ds(dstp, CHAR_CHUNK)],
                sem_o)
    kl = CHAR_NCHUNK - 1
    gat[kl].wait()
    dstp = (c * 4 + b_loc) * SEQ + off + kl * CHAR_CHUNK
    out[kl] = pltpu.async_copy(rows[kl % 2],
                               gath_out.at[pl.ds(dstp, CHAR_CHUNK)], sem_o)
    out[kl - 1].wait()
    out[kl].wait()

    @pl.when(s < 4)
    def _():
        b = c * 4 + s
        pltpu.sync_copy(shared_v.at[pl.ds(s * BUFW, SEQ)], bounce_v)
        pltpu.sync_copy(bounce_v, ids_out.at[pl.ds(b * SEQ, SEQ)])


def _rmsnorm(x, w):
    var = jnp.mean(x * x, axis=-1, keepdims=True)
    return w * (x * lax.rsqrt(var + EPS))


def _sigmoid(x):
    return 1.0 / (1.0 + jnp.exp(-x))


def _tc_params_body(temb_ref, wpre_ref, wtok_ref, wdown_ref, wproj_ref,
                    params_ref):
    f32 = jnp.float32
    te = _rmsnorm(temb_ref[...], wpre_ref[0, :])
    tf = lax.dot_general(te, wdown_ref[...], (((1,), (1,)), ((), ())),
                         precision=None, preferred_element_type=f32)
    tf = _rmsnorm(tf, wtok_ref[0, :])
    sl = tf * _sigmoid(tf)
    params_ref[0] = lax.dot_general(sl, wproj_ref[...],
                                    (((1,), (1,)), ((), ())),
                                    precision=None,
                                    preferred_element_type=f32)


def _tc_params(temb, w_pre, w_token, W_down, W_proj):
    return pl.pallas_call(
        _tc_params_body,
        grid=(B,),
        in_specs=[
            pl.BlockSpec((N_TOKEN, TOK_D), lambda b: (b, 0)),
            pl.BlockSpec((1, TOK_D), lambda b: (0, 0)),
            pl.BlockSpec((1, TEXT_D), lambda b: (0, 0)),
            pl.BlockSpec((TEXT_D, TOK_D), lambda b: (0, 0)),
            pl.BlockSpec((3, TEXT_D), lambda b: (0, 0)),
        ],
        out_specs=pl.BlockSpec((1, N_TOKEN, 3), lambda b: (b, 0, 0)),
        out_shape=jax.ShapeDtypeStruct((B, N_TOKEN, 3), jnp.float32),
        compiler_params=pltpu.CompilerParams(
            dimension_semantics=("arbitrary",)),
    )(temb, w_pre, w_token, W_down, W_proj)


def _tc_body(cnt_ref, tok_ref, params_ref, gath_ref, filler_ref,
             wchar_ref, wfus_ref, out_ref):
    b = pl.program_id(0)
    f32 = jnp.float32

    params = params_ref[0]
    tok = tok_ref[0, 0, :] >> 12
    onehot = (tok[:, None] ==
              lax.broadcasted_iota(jnp.int32, (SEQ, N_TOKEN), 1)).astype(f32)
    p_slot = lax.dot_general(onehot, params, (((1,), (0,)), ((), ())),
                             precision=None, preferred_element_type=f32)
    scale = p_slot[:, 0:1]
    shift = p_slot[:, 1:2]
    gate = p_slot[:, 2:3]

    gw = gath_ref[...]
    lo = jax.lax.bitcast_convert_type(gw << 16, f32)
    hi = jax.lax.bitcast_convert_type(gw & jnp.int32(-65536), f32)
    gath = jnp.concatenate([lo, hi], axis=1)
    cn = _rmsnorm(gath, wchar_ref[0, :])
    h = cn * (1.0 + scale) + shift
    h = h * _sigmoid(h)
    h = lax.dot_general(h, wfus_ref[...], (((1,), (1,)), ((), ())),
                        precision=None, preferred_element_type=f32)
    g = _sigmoid(gate)
    mix = g * h + (1.0 - g) * cn

    cnt = cnt_ref[b, 0]
    valid = lax.broadcasted_iota(jnp.int32, (SEQ, 1), 0) < cnt
    out_ref[...] = jnp.where(valid, mix, filler_ref[...])


def _tc_fuse(cnt, tok, params, gath, filler, w_char, W_fus):
    return pl.pallas_call(
        _tc_body,
        grid=(B,),
        in_specs=[
            pl.BlockSpec(memory_space=pltpu.SMEM),
            pl.BlockSpec((1, 1, SEQ), lambda b: (b, 0, 0)),
            pl.BlockSpec((1, N_TOKEN, 3), lambda b: (b, 0, 0)),
            pl.BlockSpec((SEQ, HALF_D), lambda b: (b, 0)),
            pl.BlockSpec((1, TEXT_D), lambda b: (0, 0)),
            pl.BlockSpec((1, TEXT_D), lambda b: (0, 0)),
            pl.BlockSpec((TEXT_D, TEXT_D), lambda b: (0, 0)),
        ],
        out_specs=pl.BlockSpec((SEQ, TEXT_D), lambda b: (b, 0)),
        out_shape=jax.ShapeDtypeStruct((B * SEQ, TEXT_D), jnp.float32),
        compiler_params=pltpu.CompilerParams(
            dimension_semantics=("arbitrary",)),
    )(cnt, tok, params, gath, filler, w_char, W_fus)


def kernel(token_ids, token_ids_mask, char_ids, char_ids_mask, seq_len,
           text_embed, token_embed, w_pre, w_token, w_char, W_down, W_proj,
           W_fus):
    del token_ids_mask, seq_len
    token_flat = token_ids.reshape(-1).astype(jnp.int32)
    cid2d = char_ids.reshape(ROWS, 128).astype(jnp.int32)
    mask2d = char_ids_mask.reshape(ROWS, 128).astype(jnp.int32)

    dest, packed, cnt, tbl = _tc_prep(mask2d, cid2d, text_embed)
    ids, gath, temb = _sc_compact_gather(token_flat,
                                         packed.reshape(B, 32, 128),
                                         dest.reshape(B, 32, 128), tbl,
                                         token_embed)
    params = _tc_params(temb, w_pre.reshape(1, TOK_D),
                        w_token.reshape(1, TEXT_D), W_down, W_proj)

    out = _tc_fuse(cnt, ids.reshape(B, 1, SEQ), params, gath,
                   text_embed[0:1], w_char.reshape(1, TEXT_D), W_fus)
    return out.reshape(B, SEQ, TEXT_D)


# speedup vs baseline: 1.1869x; 1.1869x over previous
import functools

import jax
import jax.numpy as jnp
from jax import lax
from jax.experimental import pallas as pl
from jax.experimental.pallas import tpu as pltpu
from jax.experimental.pallas import tpu_sc as plsc

B = 8
N_TOKEN = 512
N_CHAR = 8
FLAT = N_TOKEN * N_CHAR
SEQ = 2048
TEXT_D = 512
TOK_D = 768
EPS = 1e-6
ROWS = B * 32
BUFW = FLAT + 16
HALF_D = TEXT_D // 2

_info = plsc.get_sparse_core_info()
NC, NS = _info.num_cores, _info.num_subcores
NW = NC * NS

_MESH = plsc.VectorSubcoreMesh(core_axis_name="c", subcore_axis_name="s")
_HI = lax.Precision.HIGHEST


def _bf16_bits(x):
    u = jax.lax.bitcast_convert_type(x, jnp.int32)
    r = u + 0x7FFF + ((u >> 16) & 1)
    return (r >> 16) & 0xFFFF


def _tc_prep_body(mask_ref, cid_ref, text_ref, dest_ref, packed_ref, cnt_ref,
                  tbl_ref):
    f32 = jnp.float32
    i32 = jnp.int32
    mi = mask_ref[...]
    m2 = mi.astype(f32)
    a_ = lax.broadcasted_iota(i32, (ROWS, ROWS), 0)
    b_ = lax.broadcasted_iota(i32, (ROWS, ROWS), 1)
    lt_bd = ((b_ < a_) & (b_ // 32 == a_ // 32)).astype(f32)
    prev_rows = lax.dot_general(lt_bd, m2, (((1,), (0,)), ((), ())),
                                precision=_HI, preferred_element_type=f32)
    rowoff = jnp.sum(prev_rows, axis=-1, keepdims=True)
    cp = lax.broadcasted_iota(i32, (128, 128), 0)
    cc = lax.broadcasted_iota(i32, (128, 128), 1)
    lt_incl = (cp <= cc).astype(f32)
    incl = lax.dot_general(m2, lt_incl, (((1,), (0,)), ((), ())),
                           precision=_HI, preferred_element_type=f32) + rowoff
    row = lax.broadcasted_iota(i32, (ROWS, 128), 0)
    col = lax.broadcasted_iota(i32, (ROWS, 128), 1)
    base = lax.rem(row // 32, 4) * BUFW
    dest_ref[...] = base + jnp.where(mi != 0, incl.astype(i32) - 1,
                                     FLAT + lax.rem(col, 16))
    tok = (lax.rem(row, 32) * 128 + col) // N_CHAR
    packed_ref[...] = (cid_ref[...] + 1) + tok * 4096
    rowsum = jnp.sum(m2, axis=-1, keepdims=True)
    bb = lax.broadcasted_iota(i32, (B, ROWS), 0)
    rr = lax.broadcasted_iota(i32, (B, ROWS), 1)
    ind = (rr // 32 == bb).astype(f32)
    total = lax.dot_general(ind, rowsum, (((1,), (0,)), ((), ())),
                            precision=_HI, preferred_element_type=f32)
    cnt_ref[...] = jnp.broadcast_to(total.astype(i32), (B, 128))
    tx = text_ref[...]
    lo = _bf16_bits(tx[:, :HALF_D])
    hi = _bf16_bits(tx[:, HALF_D:])
    tbl_ref[...] = lo | (hi << 16)


def _tc_prep(mask2d, cid2d, text_embed):
    return pl.pallas_call(
        _tc_prep_body,
        out_shape=[
            jax.ShapeDtypeStruct((ROWS, 128), jnp.int32),
            jax.ShapeDtypeStruct((ROWS, 128), jnp.int32),
            jax.ShapeDtypeStruct((B, 128), jnp.int32),
            jax.ShapeDtypeStruct((SEQ + 1, HALF_D), jnp.int32),
        ],
    )(mask2d, cid2d, text_embed)


TOK_ROWS_PER_W = (B * N_TOKEN) // NW
TOK_CHUNK = 64
TOK_NCHUNK = TOK_ROWS_PER_W // TOK_CHUNK


@functools.partial(
    pl.kernel,
    mesh=_MESH,
    out_type=jax.ShapeDtypeStruct((B * N_TOKEN, TOK_D), jnp.float32),
    scratch_types=[
        pltpu.VMEM((TOK_CHUNK,), jnp.int32),
        pltpu.VMEM((TOK_CHUNK,), jnp.int32),
        pltpu.VMEM((TOK_CHUNK, TOK_D), jnp.float32),
        pltpu.VMEM((TOK_CHUNK, TOK_D), jnp.float32),
        pltpu.SemaphoreType.DMA,
        pltpu.SemaphoreType.DMA,
    ],
)
def _sc_token_gather(token_ids_hbm, token_embed_hbm, temb_out,
                     idx0_v, idx1_v, rows0_v, rows1_v, sem_g, sem_o):
    wid = lax.axis_index("s") * NC + lax.axis_index("c")
    idx = [idx0_v, idx1_v]
    rows = [rows0_v, rows1_v]
    gat = [None] * TOK_NCHUNK
    out = [None] * TOK_NCHUNK
    for k in range(TOK_NCHUNK):
        base = wid * TOK_ROWS_PER_W + k * TOK_CHUNK
        pltpu.sync_copy(token_ids_hbm.at[pl.ds(base, TOK_CHUNK)], idx[k % 2])
        gat[k] = pltpu.async_copy(token_embed_hbm.at[idx[k % 2]],
                                  rows[k % 2], sem_g)
    for k in range(TOK_NCHUNK):
        base = wid * TOK_ROWS_PER_W + k * TOK_CHUNK
        gat[k].wait()
        out[k] = pltpu.async_copy(rows[k % 2],
                                  temb_out.at[pl.ds(base, TOK_CHUNK)], sem_o)
    for k in range(TOK_NCHUNK):
        out[k].wait()


CHAR_CHUNK = 128
CHAR_NCHUNK = 512 // CHAR_CHUNK


@functools.partial(
    pl.kernel,
    mesh=_MESH,
    out_type=[
        jax.ShapeDtypeStruct((B * SEQ,), jnp.int32),
        jax.ShapeDtypeStruct((B * SEQ, HALF_D), jnp.int32),
    ],
    scratch_types=[
        pltpu.VMEM((32, 128), jnp.int32),
        pltpu.VMEM((32, 128), jnp.int32),
        pltpu.VMEM((CHAR_CHUNK,), jnp.int32),
        pltpu.VMEM((CHAR_CHUNK,), jnp.int32),
        pltpu.VMEM((CHAR_CHUNK, HALF_D), jnp.int32),
        pltpu.VMEM((CHAR_CHUNK, HALF_D), jnp.int32),
        pltpu.VMEM((SEQ,), jnp.int32),
        pltpu.VMEM_SHARED((4 * BUFW,), jnp.int32),
        pltpu.SemaphoreType.DMA,
        pltpu.SemaphoreType.DMA,
    ],
)
def _sc_compact_gather(packed_hbm, dest_hbm, tbl_hbm, ids_out, gath_out,
                       vals_v, dst_v, idx0_v, idx1_v, rows0_v, rows1_v,
                       bounce_v, shared_v, sem, sem_o):
    c = lax.axis_index("c")
    s = lax.axis_index("s")

    @pl.when(s < 4)
    def _():
        b = c * 4 + s
        pltpu.sync_copy(packed_hbm.at[b], vals_v)
        pltpu.sync_copy(dest_hbm.at[b], dst_v)

        def scat_body(j4, carry):
            ds = []
            for p in range(8):
                j = j4 * 8 + p
                ds.append(pltpu.async_copy(
                    vals_v.at[j], shared_v.at[dst_v.at[j]], sem))
            for d in ds:
                d.wait()
            return carry

        lax.fori_loop(0, 4, scat_body, 0)

    plsc.subcore_barrier()

    b_loc = s // 4
    off = lax.rem(s, 4) * 512
    idx = [idx0_v, idx1_v]
    rows = [rows0_v, rows1_v]
    gat = [None] * CHAR_NCHUNK
    out = [None] * CHAR_NCHUNK

    for k in range(CHAR_NCHUNK):
        if k >= 2:
            out[k - 2].wait()
        src = b_loc * BUFW + off + k * CHAR_CHUNK
        ib = idx[k % 2]
        pltpu.sync_copy(shared_v.at[pl.ds(src, CHAR_CHUNK)], ib)
        for p in range(CHAR_CHUNK // 16):
            v = ib[pl.ds(p * 16, 16)]
            ib[pl.ds(p * 16, 16)] = jnp.clip(jnp.bitwise_and(v, 4095), 0, SEQ)
        gat[k] = pltpu.async_copy(tbl_hbm.at[ib], rows[k % 2], sem)
        if k >= 1:
            gat[k - 1].wait()
            dstp = (c * 4 + b_loc) * SEQ + off + (k - 1) * CHAR_CHUNK
            out[k - 1] = pltpu.async_copy(
                rows[(k - 1) % 2], gath_out.at[pl.ds(dstp, CHAR_CHUNK)],
                sem_o)
    kl = CHAR_NCHUNK - 1
    gat[kl].wait()
    dstp = (c * 4 + b_loc) * SEQ + off + kl * CHAR_CHUNK
    out[kl] = pltpu.async_copy(rows[kl % 2],
                               gath_out.at[pl.ds(dstp, CHAR_CHUNK)], sem_o)
    out[kl - 1].wait()
    out[kl].wait()

    @pl.when(s < 4)
    def _():
        b = c * 4 + s
        pltpu.sync_copy(shared_v.at[pl.ds(s * BUFW, SEQ)], bounce_v)
        pltpu.sync_copy(bounce_v, ids_out.at[pl.ds(b * SEQ, SEQ)])


def _rmsnorm(x, w):
    var = jnp.mean(x * x, axis=-1, keepdims=True)
    return w * (x * lax.rsqrt(var + EPS))


def _sigmoid(x):
    return 1.0 / (1.0 + jnp.exp(-x))


def _tc_params_body(temb_ref, wpre_ref, wtok_ref, wdown_ref, wproj_ref,
                    params_ref):
    f32 = jnp.float32
    te = _rmsnorm(temb_ref[...], wpre_ref[0, :])
    tf = lax.dot_general(te, wdown_ref[...], (((1,), (1,)), ((), ())),
                         precision=None, preferred_element_type=f32)
    tf = _rmsnorm(tf, wtok_ref[0, :])
    sl = tf * _sigmoid(tf)
    params_ref[0] = lax.dot_general(sl, wproj_ref[...],
                                    (((1,), (1,)), ((), ())),
                                    precision=None,
                                    preferred_element_type=f32)


def _tc_params(temb, w_pre, w_token, W_down, W_proj):
    return pl.pallas_call(
        _tc_params_body,
        grid=(B,),
        in_specs=[
            pl.BlockSpec((N_TOKEN, TOK_D), lambda b: (b, 0)),
            pl.BlockSpec((1, TOK_D), lambda b: (0, 0)),
            pl.BlockSpec((1, TEXT_D), lambda b: (0, 0)),
            pl.BlockSpec((TEXT_D, TOK_D), lambda b: (0, 0)),
            pl.BlockSpec((3, TEXT_D), lambda b: (0, 0)),
        ],
        out_specs=pl.BlockSpec((1, N_TOKEN, 3), lambda b: (b, 0, 0)),
        out_shape=jax.ShapeDtypeStruct((B, N_TOKEN, 3), jnp.float32),
        compiler_params=pltpu.CompilerParams(
            dimension_semantics=("arbitrary",)),
    )(temb, w_pre, w_token, W_down, W_proj)


def _tc_body(cnt_ref, tok_ref, params_ref, gath_ref, filler_ref,
             wchar_ref, wfus_ref, out_ref):
    b = pl.program_id(0)
    f32 = jnp.float32

    params = params_ref[0]
    tok = tok_ref[0, 0, :] >> 12
    onehot = (tok[:, None] ==
              lax.broadcasted_iota(jnp.int32, (SEQ, N_TOKEN), 1)).astype(f32)
    p_slot = lax.dot_general(onehot, params, (((1,), (0,)), ((), ())),
                             precision=None, preferred_element_type=f32)
    scale = p_slot[:, 0:1]
    shift = p_slot[:, 1:2]
    gate = p_slot[:, 2:3]

    gw = gath_ref[...]
    lo = jax.lax.bitcast_convert_type(gw << 16, f32)
    hi = jax.lax.bitcast_convert_type(gw & jnp.int32(-65536), f32)
    gath = jnp.concatenate([lo, hi], axis=1)
    cn = _rmsnorm(gath, wchar_ref[0, :])
    h = cn * (1.0 + scale) + shift
    h = h * _sigmoid(h)
    h = lax.dot_general(h, wfus_ref[...], (((1,), (1,)), ((), ())),
                        precision=None, preferred_element_type=f32)
    g = _sigmoid(gate)
    mix = g * h + (1.0 - g) * cn

    cnt = cnt_ref[b, 0]
    valid = lax.broadcasted_iota(jnp.int32, (SEQ, 1), 0) < cnt
    out_ref[...] = jnp.where(valid, mix, filler_ref[...])


def _tc_fuse(cnt, tok, params, gath, filler, w_char, W_fus):
    return pl.pallas_call(
        _tc_body,
        grid=(B,),
        in_specs=[
            pl.BlockSpec(memory_space=pltpu.SMEM),
            pl.BlockSpec((1, 1, SEQ), lambda b: (b, 0, 0)),
            pl.BlockSpec((1, N_TOKEN, 3), lambda b: (b, 0, 0)),
            pl.BlockSpec((SEQ, HALF_D), lambda b: (b, 0)),
            pl.BlockSpec((1, TEXT_D), lambda b: (0, 0)),
            pl.BlockSpec((1, TEXT_D), lambda b: (0, 0)),
            pl.BlockSpec((TEXT_D, TEXT_D), lambda b: (0, 0)),
        ],
        out_specs=pl.BlockSpec((SEQ, TEXT_D), lambda b: (b, 0)),
        out_shape=jax.ShapeDtypeStruct((B * SEQ, TEXT_D), jnp.float32),
        compiler_params=pltpu.CompilerParams(
            dimension_semantics=("arbitrary",)),
    )(cnt, tok, params, gath, filler, w_char, W_fus)


def kernel(token_ids, token_ids_mask, char_ids, char_ids_mask, seq_len,
           text_embed, token_embed, w_pre, w_token, w_char, W_down, W_proj,
           W_fus):
    del token_ids_mask, seq_len
    token_flat = token_ids.reshape(-1).astype(jnp.int32)
    cid2d = char_ids.reshape(ROWS, 128).astype(jnp.int32)
    mask2d = char_ids_mask.reshape(ROWS, 128).astype(jnp.int32)

    temb = _sc_token_gather(token_flat, token_embed)
    params = _tc_params(temb, w_pre.reshape(1, TOK_D),
                        w_token.reshape(1, TEXT_D), W_down, W_proj)

    dest, packed, cnt, tbl = _tc_prep(mask2d, cid2d, text_embed)
    ids, gath = _sc_compact_gather(packed.reshape(B, 32, 128),
                                   dest.reshape(B, 32, 128), tbl)

    out = _tc_fuse(cnt, ids.reshape(B, 1, SEQ), params, gath,
                   text_embed[0:1], w_char.reshape(1, TEXT_D), W_fus)
    return out.reshape(B, SEQ, TEXT_D)
